# Initial kernel scaffold; baseline (speedup 1.0000x reference)
import jax
import jax.numpy as jnp
from jax.experimental import pallas as pl


def kernel(predicts, similarities, adj_values, adj_mask):
    raise NotImplementedError("write your pallas kernel here")


# speedup vs baseline: 2.8886x; 2.8886x over previous
import functools

import jax
import jax.numpy as jnp
from jax.experimental import pallas as pl
from jax.experimental.pallas import tpu as pltpu

B = 64
L = 4096
T = 1024
NT = L // T


def _fused_kernel(predicts_ref, sims_ref, adjv_ref, adjm_ref, out_ref,
                  contrib, colsum, diag, sumsq):
    j = pl.program_id(0)
    i = pl.program_id(1)

    @pl.when(i == 0)
    def _init_j():
        contrib[...] = jnp.zeros_like(contrib)
        colsum[...] = jnp.zeros_like(colsum)

    @pl.when(jnp.logical_and(i == 0, j == 0))
    def _init_all():
        sumsq[...] = jnp.zeros_like(sumsq)

    m = adjm_ref[...].astype(jnp.float32)
    a = adjv_ref[...] * m
    p_i = predicts_ref[:, pl.ds(i * T, T)]
    contrib[...] += jnp.dot(p_i, a, preferred_element_type=jnp.float32)
    ones8 = jnp.ones((8, T), jnp.float32)
    colsum[...] += jnp.dot(ones8, m, preferred_element_type=jnp.float32)

    @pl.when(i == j)
    def _take_diag():
        rows = jax.lax.broadcasted_iota(jnp.int32, (T, T), 0)
        cols = jax.lax.broadcasted_iota(jnp.int32, (T, T), 1)
        eye = (rows == cols).astype(jnp.float32)
        diag[...] = jnp.sum(m * eye, axis=0, keepdims=True)

    @pl.when(i == NT - 1)
    def _finish_j():
        one_minus_diag = 1.0 - diag[...]
        relation = colsum[0:1, :] + one_minus_diag
        p_j = predicts_ref[:, pl.ds(j * T, T)]
        cand = (contrib[...] + p_j * one_minus_diag) / relation
        res = p_j - jnp.dot(sims_ref[...], cand,
                            preferred_element_type=jnp.float32)
        sumsq[...] += jnp.sum(res * res, axis=1, keepdims=True)

        @pl.when(j == NT - 1)
        def _finalize():
            norms = jnp.sqrt(sumsq[...])
            rowsum = jnp.sum(sims_ref[...], axis=1, keepdims=True)
            valid = (rowsum != 0.0).astype(jnp.float32)
            loss = jnp.sum(norms * valid) / jnp.sum(valid)
            out_ref[...] = jnp.reshape(loss, (1, 1))


@functools.partial(jax.jit, static_argnames=("interpret",))
def _run(predicts, similarities, adj_values, adj_mask, interpret=False):
    out = pl.pallas_call(
        _fused_kernel,
        grid=(NT, NT),
        in_specs=[
            pl.BlockSpec((B, L), lambda j, i: (0, 0)),
            pl.BlockSpec((B, B), lambda j, i: (0, 0)),
            pl.BlockSpec((T, T), lambda j, i: (i, j)),
            pl.BlockSpec((T, T), lambda j, i: (i, j)),
        ],
        out_specs=pl.BlockSpec((1, 1), lambda j, i: (0, 0)),
        out_shape=jax.ShapeDtypeStruct((1, 1), jnp.float32),
        scratch_shapes=[
            pltpu.VMEM((B, T), jnp.float32),
            pltpu.VMEM((8, T), jnp.float32),
            pltpu.VMEM((1, T), jnp.float32),
            pltpu.VMEM((B, 1), jnp.float32),
        ],
        interpret=interpret,
    )(predicts, similarities, adj_values, adj_mask)
    return out[0, 0]


def kernel(predicts, similarities, adj_values, adj_mask):
    return _run(predicts, similarities, adj_values, adj_mask)


# speedup vs baseline: 2.9179x; 1.0101x over previous
import functools

import jax
import jax.numpy as jnp
from jax.experimental import pallas as pl
from jax.experimental.pallas import tpu as pltpu

B = 64
L = 4096
TI = 256
NI = L // TI


def _fused_kernel(predicts_ref, sims_ref, adjv_ref, adjm_ref, out_ref,
                  contrib, colsum, diag):
    i = pl.program_id(0)

    @pl.when(i == 0)
    def _init():
        contrib[...] = jnp.zeros_like(contrib)
        colsum[...] = jnp.zeros_like(colsum)

    m = adjm_ref[...].astype(jnp.float32)
    a = adjv_ref[...] * m
    p_i = predicts_ref[:, pl.ds(i * TI, TI)]
    contrib[...] += jnp.dot(p_i, a, preferred_element_type=jnp.float32)
    ones8 = jnp.ones((8, TI), jnp.float32)
    colsum[...] += jnp.dot(ones8, m, preferred_element_type=jnp.float32)

    m_sq = adjm_ref[:, pl.ds(i * TI, TI)].astype(jnp.float32)
    rows = jax.lax.broadcasted_iota(jnp.int32, (TI, TI), 0)
    cols = jax.lax.broadcasted_iota(jnp.int32, (TI, TI), 1)
    eye = (rows == cols).astype(jnp.float32)
    diag[0:1, pl.ds(i * TI, TI)] = jnp.sum(m_sq * eye, axis=0,
                                           keepdims=True)

    @pl.when(i == NI - 1)
    def _finalize():
        one_minus_diag = 1.0 - diag[...]
        relation = colsum[0:1, :] + one_minus_diag
        p = predicts_ref[...]
        cand = (contrib[...] + p * one_minus_diag) / relation
        res = p - jnp.dot(sims_ref[...], cand,
                          preferred_element_type=jnp.float32)
        sumsq = jnp.sum(res * res, axis=1, keepdims=True)
        norms = jnp.sqrt(sumsq)
        rowsum = jnp.sum(sims_ref[...], axis=1, keepdims=True)
        valid = (rowsum != 0.0).astype(jnp.float32)
        loss = jnp.sum(norms * valid) / jnp.sum(valid)
        out_ref[...] = jnp.reshape(loss, (1, 1))


@functools.partial(jax.jit, static_argnames=("interpret",))
def _run(predicts, similarities, adj_values, adj_mask, interpret=False):
    out = pl.pallas_call(
        _fused_kernel,
        grid=(NI,),
        in_specs=[
            pl.BlockSpec((B, L), lambda i: (0, 0)),
            pl.BlockSpec((B, B), lambda i: (0, 0)),
            pl.BlockSpec((TI, L), lambda i: (i, 0)),
            pl.BlockSpec((TI, L), lambda i: (i, 0)),
        ],
        out_specs=pl.BlockSpec((1, 1), lambda i: (0, 0)),
        out_shape=jax.ShapeDtypeStruct((1, 1), jnp.float32),
        scratch_shapes=[
            pltpu.VMEM((B, L), jnp.float32),
            pltpu.VMEM((8, L), jnp.float32),
            pltpu.VMEM((1, L), jnp.float32),
        ],
        interpret=interpret,
    )(predicts, similarities, adj_values, adj_mask)
    return out[0, 0]


def kernel(predicts, similarities, adj_values, adj_mask):
    return _run(predicts, similarities, adj_values, adj_mask)
